# baseline (speedup 1.0000x reference)
import functools

import jax
import jax.numpy as jnp
from jax import lax
from jax.experimental import pallas as pl
from jax.experimental.pallas import tpu as pltpu
from jax.experimental.pallas import tpu_sc as plsc

_INFO = plsc.get_sparse_core_info()
_NC = _INFO.num_cores
_NS = _INFO.num_subcores
_NW = _NC * _NS
_L = 16
_VB = 2048


def _tc_convert(table_t):
    embed, vocab = table_t.shape
    grid = (vocab + _VB - 1) // _VB

    def body(in_ref, out_ref):
        out_ref[...] = in_ref[...].T

    return pl.pallas_call(
        body,
        grid=(grid,),
        in_specs=[pl.BlockSpec((embed, _VB), lambda i: (0, i))],
        out_specs=pl.BlockSpec((_VB, embed), lambda i: (i, 0)),
        out_shape=jax.ShapeDtypeStruct((vocab, embed), jnp.float32),
    )(table_t)


def _gather_impl(table, idx2d, batch, seq, embed):
    bh_n = batch // 128
    n_slabs = seq * bh_n
    per_w = n_slabs // _NW
    eh_n = embed // 8

    mesh = plsc.VectorSubcoreMesh(core_axis_name="c", subcore_axis_name="s")

    @functools.partial(
        pl.kernel,
        mesh=mesh,
        compiler_params=pltpu.CompilerParams(
            use_tc_tiling_on_sc=False, needs_layout_passes=False
        ),
        out_type=jax.ShapeDtypeStruct(
            (seq, eh_n, bh_n, 8, 128), jnp.float32
        ),
        scratch_types=[
            pltpu.VMEM((per_w, 128), jnp.int32),
            pltpu.VMEM((128, embed), jnp.float32),
            pltpu.VMEM((128, embed), jnp.float32),
            pltpu.VMEM((embed, 128), jnp.float32),
            pltpu.VMEM((embed, 128), jnp.float32),
            pltpu.SemaphoreType.DMA,
            pltpu.SemaphoreType.DMA,
            pltpu.SemaphoreType.DMA,
            pltpu.SemaphoreType.DMA,
        ],
    )
    def k(table_hbm, idx_hbm, out_hbm, idx_all, rows0, rows1, slab0, slab1,
          sem_g0, sem_g1, sem_o0, sem_o1):
        rows = (rows0, rows1)
        slab = (slab0, slab1)
        sem_g = (sem_g0, sem_g1)
        sem_o = (sem_o0, sem_o1)
        wid = lax.axis_index("s") * _NC + lax.axis_index("c")
        base = wid * per_w

        pltpu.sync_copy(
            idx_hbm.at[pl.ds(pl.multiple_of(base, 8), per_w)], idx_all
        )

        lanes = lax.iota(jnp.int32, _L)

        def fire_gather(j_local, b):
            pltpu.async_copy(
                table_hbm.at[idx_all.at[j_local]], rows[b], sem_g[b]
            )

        def transpose(b):
            def brow(r0, carry):
                for k in range(_L):
                    col = jnp.full((_L,), 0, jnp.int32) + (r0 + k)
                    for c in range(embed // _L):
                        v = rows[b][r0 + k, pl.ds(c * _L, _L)]
                        plsc.store_scatter(
                            slab[b], [lanes + c * _L, col], v
                        )
                return carry
            lax.fori_loop(0, 128 // _L, lambda i, c: brow(i * _L, c), 0)

        def fire_store(j, b):
            s = j // bh_n
            bh = j % bh_n
            for eh in range(eh_n):
                pltpu.async_copy(
                    slab[b].at[pl.ds(eh * 8, 8)],
                    out_hbm.at[s, eh, bh],
                    sem_o[b],
                )

        def drain_store(b):
            pltpu.make_async_copy(
                out_hbm.at[0, 0, 0], slab[b].at[pl.ds(0, 8)], sem_o[b]
            ).wait()

        def drain_gather_full(b):
            pltpu.make_async_copy(
                table_hbm.at[pl.ds(0, 128)], rows[b], sem_g[b]
            ).wait()

        def drain_store_full(b):
            for _ in range(eh_n):
                drain_store(b)

        fire_gather(0, 0)

        def step(i, carry):
            for u in range(2):
                i2 = i * 2 + u
                jn = jnp.minimum(i2 + 1, per_w - 1)
                fire_gather(jn, (u + 1) % 2)

                @pl.when(i2 >= 2)
                def _():
                    drain_store_full(u)

                drain_gather_full(u)
                transpose(u)
                fire_store(base + i2, u)
            return carry

        lax.fori_loop(0, per_w // 2, step, 0)

        drain_gather_full(0)
        drain_store_full(0)
        drain_store_full(1)

    return k(table, idx2d)


def kernel(words, feats, table):
    batch, seq = words.shape
    vocab, embed = table.shape
    tconv = _tc_convert(table.T)
    idx2d = words.T.reshape(seq * batch // 128, 128)
    x = _gather_impl(tconv, idx2d, batch, seq, embed)
    return x.transpose(2, 4, 0, 1, 3).reshape(batch, seq, embed)


# speedup vs baseline: 1.2406x; 1.2406x over previous
import functools

import jax
import jax.numpy as jnp
from jax import lax
from jax.experimental import pallas as pl
from jax.experimental.pallas import tpu as pltpu
from jax.experimental.pallas import tpu_sc as plsc

_INFO = plsc.get_sparse_core_info()
_NC = _INFO.num_cores
_NS = _INFO.num_subcores
_NW = _NC * _NS

_IW = 128
_G = 2
_NBUF = 4
_VB = 2048


def _tc_convert(table_t):
    embed, vocab = table_t.shape
    grid = (vocab + _VB - 1) // _VB

    def body(in_ref, out_ref):
        out_ref[...] = in_ref[...].T

    return pl.pallas_call(
        body,
        grid=(grid,),
        in_specs=[pl.BlockSpec((embed, _VB), lambda i: (0, i))],
        out_specs=pl.BlockSpec((_VB, embed), lambda i: (i, 0)),
        out_shape=jax.ShapeDtypeStruct((vocab, embed), jnp.float32),
    )(table_t)


def _gather_impl(table, idx2d, n_rows, embed):
    n_per_w = n_rows // _NW
    iw_per_w = n_per_w // _IW
    blk = _G * _IW
    n_blocks = n_per_w // blk
    assert n_blocks % _NBUF == 0 and n_blocks * blk == n_per_w

    mesh = plsc.VectorSubcoreMesh(core_axis_name="c", subcore_axis_name="s")

    @functools.partial(
        pl.kernel,
        mesh=mesh,
        compiler_params=pltpu.CompilerParams(use_tc_tiling_on_sc=False),
        out_type=jax.ShapeDtypeStruct((n_rows, embed), jnp.float32),
        scratch_types=[
            pltpu.VMEM((iw_per_w, _IW), jnp.int32),
            *([pltpu.VMEM((blk, embed), jnp.float32)] * _NBUF),
            *([pltpu.SemaphoreType.DMA] * (2 * _NBUF)),
        ],
    )
    def k(table_hbm, idx_hbm, out_hbm, idx_all, *bufs_and_sems):
        rows = bufs_and_sems[:_NBUF]
        sem_g = bufs_and_sems[_NBUF:2 * _NBUF]
        sem_o = bufs_and_sems[2 * _NBUF:]
        wid = lax.axis_index("s") * _NC + lax.axis_index("c")
        row_base = wid * n_per_w

        pltpu.sync_copy(
            idx_hbm.at[pl.ds(pl.multiple_of(wid * iw_per_w, 8), iw_per_w)],
            idx_all,
        )

        def fire_gathers(i, b):
            for t in range(_G):
                pltpu.async_copy(
                    table_hbm.at[idx_all.at[i * _G + t]],
                    rows[b].at[pl.ds(t * _IW, _IW)],
                    sem_g[b],
                )

        def drain_gathers(b):
            pltpu.make_async_copy(
                out_hbm.at[pl.ds(0, blk)], rows[b], sem_g[b]
            ).wait()

        def fire_store(i, b):
            row_off = pl.multiple_of(row_base + i * blk, 8)
            pltpu.async_copy(rows[b], out_hbm.at[pl.ds(row_off, blk)], sem_o[b])

        def drain_store(b):
            pltpu.make_async_copy(
                out_hbm.at[pl.ds(0, blk)], rows[b], sem_o[b]
            ).wait()

        def outer(jj, carry):
            for u in range(_NBUF):
                i = jj * _NBUF + u
                b = u

                @pl.when(jj > 0)
                def _():
                    drain_store(b)

                fire_gathers(i, b)

                b2 = (u - 2) % _NBUF
                if u >= 2:
                    drain_gathers(b2)
                    fire_store(i - 2, b2)
                else:
                    @pl.when(jj > 0)
                    def _():
                        drain_gathers(b2)
                        fire_store(i - 2, b2)
            return carry

        lax.fori_loop(0, n_blocks // _NBUF, outer, 0)

        for i in (n_blocks - 2, n_blocks - 1):
            b = i % _NBUF
            drain_gathers(b)
            fire_store(i, b)
        for b in range(_NBUF):
            drain_store(b)

    return k(table, idx2d)


def kernel(words, feats, table):
    batch, seq = words.shape
    vocab, embed = table.shape
    n_rows = batch * seq
    idx2d = words.reshape(n_rows // _IW, _IW)
    out = _gather_impl(_tc_convert(table.T), idx2d, n_rows, embed)
    return out.reshape(batch, seq, embed)
